# baseline (speedup 1.0000x reference)
import functools

import jax
import jax.numpy as jnp
from jax import lax
from jax.experimental import pallas as pl
from jax.experimental.pallas import tpu as pltpu
from jax.experimental.pallas import tpu_sc as plsc

_LANES = 16


def _gram_body(n, logits_ref, labels_ref, out_ref, h_ref):
    lab = labels_ref[...]
    lgt = logits_ref[...]
    ann = (lab == 0.0) | (lab == 1.0)
    y = jnp.where(ann, 2.0 * lab - 1.0, 2.0 * jax.nn.sigmoid(lgt) - 1.0)
    h = lax.dot_general(y, y, (((0,), (0,)), ((), ())),
                        preferred_element_type=jnp.float32)

    @pl.when(pl.program_id(0) == 0)
    def _():
        h_ref[...] = jnp.zeros_like(h_ref)

    h_ref[...] += h

    @pl.when(pl.program_id(0) == pl.num_programs(0) - 1)
    def _():
        hf = h_ref[...]
        out_ref[...] = (hf[0:n, 0:n] + hf[n:2 * n, n:2 * n]
                        + hf[2 * n:3 * n, 2 * n:3 * n]
                        + hf[3 * n:4 * n, 3 * n:4 * n])


@functools.lru_cache(maxsize=None)
def _make_gram(rows, n, block):
    grid = rows // block
    return pl.pallas_call(
        functools.partial(_gram_body, n),
        grid=(grid,),
        in_specs=[
            pl.BlockSpec((block, 4 * n), lambda i: (i, 0)),
            pl.BlockSpec((block, 4 * n), lambda i: (i, 0)),
        ],
        out_specs=pl.BlockSpec((n, n), lambda i: (0, 0)),
        out_shape=jax.ShapeDtypeStruct((n, n), jnp.float32),
        scratch_shapes=[pltpu.VMEM((4 * n, 4 * n), jnp.float32)],
    )


@functools.lru_cache(maxsize=None)
def _make_edge_reduce(n, e, scale):
    e_pad = ((e + _LANES - 1) // _LANES) * _LANES
    chunks = e_pad // _LANES
    mesh = plsc.VectorSubcoreMesh(core_axis_name="c", subcore_axis_name="s")

    @functools.partial(
        pl.kernel,
        mesh=mesh,
        out_type=jax.ShapeDtypeStruct((_LANES,), jnp.float32),
        compiler_params=pltpu.CompilerParams(needs_layout_passes=False),
        scratch_types=[
            pltpu.VMEM((n * n,), jnp.float32),
            pltpu.VMEM((3 * e_pad,), jnp.float32),
            pltpu.VMEM((_LANES,), jnp.float32),
            pltpu.SemaphoreType.DMA,
            pltpu.SemaphoreType.DMA,
        ],
    )
    def k(g_hbm, edges_hbm, out_hbm, g_v, ed_v, o_v, sem0, sem1):
        wid = lax.axis_index("s") * 2 + lax.axis_index("c")

        @pl.when(wid == 0)
        def _():
            cp0 = pltpu.async_copy(g_hbm, g_v, sem0)
            cp1 = pltpu.async_copy(edges_hbm, ed_v, sem1)
            cp0.wait()
            cp1.wait()

            def step(c, acc):
                sl = pl.ds(c * _LANES, _LANES)
                li = plsc.bitcast(ed_v[sl], jnp.int32)
                ri = plsc.bitcast(ed_v[pl.ds(e_pad + c * _LANES, _LANES)],
                                  jnp.int32)
                we = ed_v[pl.ds(2 * e_pad + c * _LANES, _LANES)]
                gll = plsc.load_gather(g_v, [li * (n + 1)])
                grr = plsc.load_gather(g_v, [ri * (n + 1)])
                glr = plsc.load_gather(g_v, [li * n + ri])
                return acc + we * (gll + grr - 2.0 * glr)

            acc = lax.fori_loop(0, chunks, step, jnp.zeros((_LANES,),
                                                           jnp.float32))
            total = jnp.sum(acc) * scale
            o_v[...] = jnp.full((_LANES,), total, jnp.float32)
            pltpu.sync_copy(o_v, out_hbm)

    return k


def kernel(logits, labels, edge_weights, left_labels, right_labels):
    batch, n = logits.shape
    e = left_labels.shape[0]
    e_pad = ((e + _LANES - 1) // _LANES) * _LANES
    pad = e_pad - e
    z_logits = logits.reshape(batch // 4, 4 * n)
    z_labels = labels.reshape(batch // 4, 4 * n)
    g = _make_gram(batch // 4, n, 512)(z_logits, z_labels)
    edges = jnp.concatenate([
        jnp.pad(lax.bitcast_convert_type(left_labels, jnp.float32), (0, pad)),
        jnp.pad(lax.bitcast_convert_type(right_labels, jnp.float32), (0, pad)),
        jnp.pad(edge_weights, (0, pad)),
    ])
    scale = 0.1 / (batch * e)
    out = _make_edge_reduce(n, e, scale)(g.reshape(n * n), edges)
    return out[0]


# speedup vs baseline: 1.9632x; 1.9632x over previous
import functools

import jax
import jax.numpy as jnp
from jax import lax
from jax.experimental import pallas as pl
from jax.experimental.pallas import tpu as pltpu
from jax.experimental.pallas import tpu_sc as plsc

_LANES = 16


def _gram_body(n, lt_ref, bt_ref, out_ref, acc_ref):
    lab = bt_ref[...]
    lgt = lt_ref[...]
    ann = (lab == 0.0) | (lab == 1.0)
    y = jnp.where(ann, 2.0 * lab - 1.0, 2.0 * jax.nn.sigmoid(lgt) - 1.0)
    g = lax.dot_general(y, y, (((1,), (1,)), ((), ())),
                        preferred_element_type=jnp.float32)

    @pl.when(pl.program_id(0) == 0)
    def _():
        acc_ref[...] = jnp.zeros_like(acc_ref)

    acc_ref[...] += g

    @pl.when(pl.program_id(0) == pl.num_programs(0) - 1)
    def _():
        gf = acc_ref[...]
        per_row = 128 // n
        for q in range(n * n // 128):
            for m in range(per_row):
                out_ref[q:q + 1, pl.ds(n * m, n)] = gf[per_row * q + m:
                                                       per_row * q + m + 1, :]


@functools.lru_cache(maxsize=None)
def _make_gram(batch, n, block):
    grid = batch // block
    return pl.pallas_call(
        functools.partial(_gram_body, n),
        grid=(grid,),
        in_specs=[
            pl.BlockSpec((n, block), lambda i: (0, i)),
            pl.BlockSpec((n, block), lambda i: (0, i)),
        ],
        out_specs=pl.BlockSpec((n * n // 128, 128), lambda i: (0, 0)),
        out_shape=jax.ShapeDtypeStruct((n * n // 128, 128), jnp.float32),
        scratch_shapes=[pltpu.VMEM((n, n), jnp.float32)],
    )


@functools.lru_cache(maxsize=None)
def _make_edge_reduce(n, e, scale):
    e_pad = ((e + _LANES - 1) // _LANES) * _LANES
    chunks = e_pad // _LANES
    mesh = plsc.VectorSubcoreMesh(core_axis_name="c", subcore_axis_name="s")

    @functools.partial(
        pl.kernel,
        mesh=mesh,
        out_type=jax.ShapeDtypeStruct((_LANES,), jnp.float32),
        compiler_params=pltpu.CompilerParams(needs_layout_passes=False),
        scratch_types=[
            pltpu.VMEM((n * n,), jnp.float32),
            pltpu.VMEM((3 * e_pad,), jnp.float32),
            pltpu.VMEM((_LANES,), jnp.float32),
            pltpu.SemaphoreType.DMA,
            pltpu.SemaphoreType.DMA,
        ],
    )
    def k(g_hbm, edges_hbm, out_hbm, g_v, ed_v, o_v, sem0, sem1):
        wid = lax.axis_index("s") * 2 + lax.axis_index("c")

        @pl.when(wid == 0)
        def _():
            cp0 = pltpu.async_copy(g_hbm, g_v, sem0)
            cp1 = pltpu.async_copy(edges_hbm, ed_v, sem1)
            cp0.wait()
            cp1.wait()

            def step(c, acc):
                sl = pl.ds(c * _LANES, _LANES)
                li = plsc.bitcast(ed_v[sl], jnp.int32)
                ri = plsc.bitcast(ed_v[pl.ds(e_pad + c * _LANES, _LANES)],
                                  jnp.int32)
                we = ed_v[pl.ds(2 * e_pad + c * _LANES, _LANES)]
                gll = plsc.load_gather(g_v, [li * (n + 1)])
                grr = plsc.load_gather(g_v, [ri * (n + 1)])
                glr = plsc.load_gather(g_v, [li * n + ri])
                return acc + we * (gll + grr - 2.0 * glr)

            acc = lax.fori_loop(0, chunks, step, jnp.zeros((_LANES,),
                                                           jnp.float32))
            total = jnp.sum(acc) * scale
            o_v[...] = jnp.full((_LANES,), total, jnp.float32)
            pltpu.sync_copy(o_v, out_hbm)

    return k


def kernel(logits, labels, edge_weights, left_labels, right_labels):
    batch, n = logits.shape
    e = left_labels.shape[0]
    e_pad = ((e + _LANES - 1) // _LANES) * _LANES
    pad = e_pad - e
    lt = pltpu.with_memory_space_constraint(logits.T, pltpu.MemorySpace.HBM)
    bt = pltpu.with_memory_space_constraint(labels.T, pltpu.MemorySpace.HBM)
    g = _make_gram(batch, n, 2048)(lt, bt)
    edges = jnp.concatenate([
        jnp.pad(lax.bitcast_convert_type(left_labels, jnp.float32), (0, pad)),
        jnp.pad(lax.bitcast_convert_type(right_labels, jnp.float32), (0, pad)),
        jnp.pad(edge_weights, (0, pad)),
    ])
    scale = 0.1 / (batch * e)
    out = _make_edge_reduce(n, e, scale)(g.reshape(n * n), edges)
    return out[0]


# speedup vs baseline: 2.5482x; 1.2980x over previous
import functools

import jax
import jax.numpy as jnp
from jax import lax
from jax.experimental import pallas as pl
from jax.experimental.pallas import tpu as pltpu
from jax.experimental.pallas import tpu_sc as plsc

_LANES = 16


def _gram_body(n, lt_ref, bt_ref, out_ref, acc_ref):
    lab = bt_ref[...]
    lgt = lt_ref[...]
    ann = (lab == 0.0) | (lab == 1.0)
    y = jnp.where(ann, 2.0 * lab - 1.0, 2.0 * jax.nn.sigmoid(lgt) - 1.0)
    g = lax.dot_general(y, y, (((1,), (1,)), ((), ())),
                        preferred_element_type=jnp.float32)

    @pl.when(pl.program_id(0) == 0)
    def _():
        acc_ref[...] = jnp.zeros_like(acc_ref)

    acc_ref[...] += g

    @pl.when(pl.program_id(0) == pl.num_programs(0) - 1)
    def _():
        gf = acc_ref[...]
        per_row = 128 // n
        for q in range(n * n // 128):
            for m in range(per_row):
                out_ref[q:q + 1, pl.ds(n * m, n)] = gf[per_row * q + m:
                                                       per_row * q + m + 1, :]


@functools.lru_cache(maxsize=None)
def _make_gram(batch, n, block):
    grid = batch // block
    return pl.pallas_call(
        functools.partial(_gram_body, n),
        grid=(grid,),
        in_specs=[
            pl.BlockSpec((n, block), lambda i: (0, i)),
            pl.BlockSpec((n, block), lambda i: (0, i)),
        ],
        out_specs=pl.BlockSpec((n * n // 128, 128), lambda i: (0, 0)),
        out_shape=jax.ShapeDtypeStruct((n * n // 128, 128), jnp.float32),
        scratch_shapes=[pltpu.VMEM((n, n), jnp.float32)],
    )


@functools.lru_cache(maxsize=None)
def _make_edge_reduce(n, e, scale):
    e_pad = ((e + _LANES - 1) // _LANES) * _LANES
    chunks = e_pad // _LANES
    mesh = plsc.VectorSubcoreMesh(core_axis_name="c", subcore_axis_name="s",
                                  num_cores=1)

    @functools.partial(
        pl.kernel,
        mesh=mesh,
        out_type=jax.ShapeDtypeStruct((_LANES,), jnp.float32),
        compiler_params=pltpu.CompilerParams(needs_layout_passes=False),
        scratch_types=[
            pltpu.VMEM((n * n,), jnp.float32),
            pltpu.VMEM((e_pad,), jnp.int32),
            pltpu.VMEM((e_pad,), jnp.int32),
            pltpu.VMEM((e_pad,), jnp.float32),
            pltpu.VMEM((_LANES,), jnp.float32),
            pltpu.SemaphoreType.DMA,
            pltpu.SemaphoreType.DMA,
            pltpu.SemaphoreType.DMA,
            pltpu.SemaphoreType.DMA,
        ],
    )
    def k(g_hbm, l_hbm, r_hbm, w_hbm, out_hbm, g_v, l_v, r_v, w_v, o_v,
          sem0, sem1, sem2, sem3):
        wid = lax.axis_index("s")

        @pl.when(wid == 0)
        def _():
            tail = pl.ds(e_pad - _LANES, _LANES)
            l_v[tail] = jnp.zeros((_LANES,), jnp.int32)
            r_v[tail] = jnp.zeros((_LANES,), jnp.int32)
            w_v[tail] = jnp.zeros((_LANES,), jnp.float32)
            cp0 = pltpu.async_copy(g_hbm, g_v, sem0)
            cp1 = pltpu.async_copy(l_hbm, l_v.at[pl.ds(0, e)], sem1)
            cp2 = pltpu.async_copy(r_hbm, r_v.at[pl.ds(0, e)], sem2)
            cp3 = pltpu.async_copy(w_hbm, w_v.at[pl.ds(0, e)], sem3)
            cp0.wait()
            cp1.wait()
            cp2.wait()
            cp3.wait()

            def step(c, acc):
                sl = pl.ds(c * _LANES, _LANES)
                li = l_v[sl]
                ri = r_v[sl]
                we = w_v[sl]
                gll = plsc.load_gather(g_v, [li * (n + 1)])
                grr = plsc.load_gather(g_v, [ri * (n + 1)])
                glr = plsc.load_gather(g_v, [li * n + ri])
                return acc + we * (gll + grr - 2.0 * glr)

            acc = lax.fori_loop(0, chunks, step, jnp.zeros((_LANES,),
                                                           jnp.float32))
            total = jnp.sum(acc) * scale
            o_v[...] = jnp.full((_LANES,), total, jnp.float32)
            pltpu.sync_copy(o_v, out_hbm)

    return k


def kernel(logits, labels, edge_weights, left_labels, right_labels):
    batch, n = logits.shape
    e = left_labels.shape[0]
    e_pad = ((e + _LANES - 1) // _LANES) * _LANES
    pad = e_pad - e
    lt = pltpu.with_memory_space_constraint(logits.T, pltpu.MemorySpace.HBM)
    bt = pltpu.with_memory_space_constraint(labels.T, pltpu.MemorySpace.HBM)
    g = _make_gram(batch, n, 4096)(lt, bt)
    scale = 0.1 / (batch * e)
    out = _make_edge_reduce(n, e, scale)(g.reshape(n * n), left_labels,
                                         right_labels, edge_weights)
    return out[0]


# speedup vs baseline: 2.5533x; 1.0020x over previous
import functools

import jax
import jax.numpy as jnp
from jax import lax
from jax.experimental import pallas as pl
from jax.experimental.pallas import tpu as pltpu
from jax.experimental.pallas import tpu_sc as plsc

_LANES = 16


def _gram_body(n, lt_ref, bt_ref, out_ref, acc_ref):
    lab = bt_ref[...]
    lgt = lt_ref[...]
    ann = (lab == 0.0) | (lab == 1.0)
    y = jnp.where(ann, 2.0 * lab - 1.0, 2.0 * jax.nn.sigmoid(lgt) - 1.0)
    g = lax.dot_general(y, y, (((1,), (1,)), ((), ())),
                        preferred_element_type=jnp.float32)

    @pl.when(pl.program_id(0) == 0)
    def _():
        acc_ref[...] = jnp.zeros_like(acc_ref)

    acc_ref[...] += g

    @pl.when(pl.program_id(0) == pl.num_programs(0) - 1)
    def _():
        gf = acc_ref[...]
        per_row = 128 // n
        for q in range(n * n // 128):
            for m in range(per_row):
                out_ref[q:q + 1, pl.ds(n * m, n)] = gf[per_row * q + m:
                                                       per_row * q + m + 1, :]


@functools.lru_cache(maxsize=None)
def _make_gram(batch, n, block):
    grid = batch // block
    return pl.pallas_call(
        functools.partial(_gram_body, n),
        grid=(grid,),
        in_specs=[
            pl.BlockSpec((n, block), lambda i: (0, i)),
            pl.BlockSpec((n, block), lambda i: (0, i)),
        ],
        out_specs=pl.BlockSpec((n * n // 128, 128), lambda i: (0, 0)),
        out_shape=jax.ShapeDtypeStruct((n * n // 128, 128), jnp.float32),
        scratch_shapes=[pltpu.VMEM((n, n), jnp.float32)],
    )


@functools.lru_cache(maxsize=None)
def _make_edge_reduce(n, e, scale):
    e_pad = ((e + _LANES - 1) // _LANES) * _LANES
    chunks = e_pad // _LANES
    mesh = plsc.VectorSubcoreMesh(core_axis_name="c", subcore_axis_name="s",
                                  num_cores=1)

    @functools.partial(
        pl.kernel,
        mesh=mesh,
        out_type=jax.ShapeDtypeStruct((_LANES,), jnp.float32),
        compiler_params=pltpu.CompilerParams(needs_layout_passes=False,
                                             skip_device_barrier=True),
        scratch_types=[
            pltpu.VMEM((n * n,), jnp.float32),
            pltpu.VMEM((e_pad,), jnp.int32),
            pltpu.VMEM((e_pad,), jnp.int32),
            pltpu.VMEM((e_pad,), jnp.float32),
            pltpu.VMEM((_LANES,), jnp.float32),
            pltpu.SemaphoreType.DMA,
            pltpu.SemaphoreType.DMA,
            pltpu.SemaphoreType.DMA,
            pltpu.SemaphoreType.DMA,
        ],
    )
    def k(g_hbm, l_hbm, r_hbm, w_hbm, out_hbm, g_v, l_v, r_v, w_v, o_v,
          sem0, sem1, sem2, sem3):
        wid = lax.axis_index("s")

        @pl.when(wid == 0)
        def _():
            tail = pl.ds(e_pad - _LANES, _LANES)
            l_v[tail] = jnp.zeros((_LANES,), jnp.int32)
            r_v[tail] = jnp.zeros((_LANES,), jnp.int32)
            w_v[tail] = jnp.zeros((_LANES,), jnp.float32)
            cp0 = pltpu.async_copy(g_hbm, g_v, sem0)
            cp1 = pltpu.async_copy(l_hbm, l_v.at[pl.ds(0, e)], sem1)
            cp2 = pltpu.async_copy(r_hbm, r_v.at[pl.ds(0, e)], sem2)
            cp3 = pltpu.async_copy(w_hbm, w_v.at[pl.ds(0, e)], sem3)
            cp0.wait()
            cp1.wait()
            cp2.wait()
            cp3.wait()

            def step(c, acc):
                sl = pl.ds(c * _LANES, _LANES)
                li = l_v[sl]
                ri = r_v[sl]
                we = w_v[sl]
                gll = plsc.load_gather(g_v, [li * (n + 1)])
                grr = plsc.load_gather(g_v, [ri * (n + 1)])
                glr = plsc.load_gather(g_v, [li * n + ri])
                return acc + we * (gll + grr - 2.0 * glr)

            acc = lax.fori_loop(0, chunks, step, jnp.zeros((_LANES,),
                                                           jnp.float32))
            total = jnp.sum(acc) * scale
            o_v[...] = jnp.full((_LANES,), total, jnp.float32)
            pltpu.sync_copy(o_v, out_hbm)

    return k


def kernel(logits, labels, edge_weights, left_labels, right_labels):
    batch, n = logits.shape
    e = left_labels.shape[0]
    e_pad = ((e + _LANES - 1) // _LANES) * _LANES
    pad = e_pad - e
    lt = pltpu.with_memory_space_constraint(logits.T, pltpu.MemorySpace.HBM)
    bt = pltpu.with_memory_space_constraint(labels.T, pltpu.MemorySpace.HBM)
    g = _make_gram(batch, n, 4096)(lt, bt)
    scale = 0.1 / (batch * e)
    out = _make_edge_reduce(n, e, scale)(g.reshape(n * n), left_labels,
                                         right_labels, edge_weights)
    return out[0]


# speedup vs baseline: 2.6674x; 1.0447x over previous
import functools

import jax
import jax.numpy as jnp
from jax import lax
from jax.experimental import pallas as pl
from jax.experimental.pallas import tpu as pltpu
from jax.experimental.pallas import tpu_sc as plsc

_LANES = 16


def _gram_body(n, lt_ref, bt_ref, out_ref, acc_ref):
    lab = bt_ref[...]
    lgt = lt_ref[...]
    ann = (lab == 0.0) | (lab == 1.0)
    y = jnp.where(ann, 2.0 * lab - 1.0, 2.0 * jax.nn.sigmoid(lgt) - 1.0)
    g = lax.dot_general(y, y, (((1,), (1,)), ((), ())),
                        preferred_element_type=jnp.float32)

    @pl.when(pl.program_id(0) == 0)
    def _():
        acc_ref[...] = jnp.zeros_like(acc_ref)

    acc_ref[...] += g

    @pl.when(pl.program_id(0) == pl.num_programs(0) - 1)
    def _():
        gf = acc_ref[...]
        per_row = 128 // n
        for q in range(n * n // 128):
            for m in range(per_row):
                out_ref[q:q + 1, pl.ds(n * m, n)] = gf[per_row * q + m:
                                                       per_row * q + m + 1, :]


@functools.lru_cache(maxsize=None)
def _make_gram(batch, n, block):
    grid = batch // block
    return pl.pallas_call(
        functools.partial(_gram_body, n),
        grid=(grid,),
        in_specs=[
            pl.BlockSpec((n, block), lambda i: (0, i)),
            pl.BlockSpec((n, block), lambda i: (0, i)),
        ],
        out_specs=pl.BlockSpec((n * n // 128, 128), lambda i: (0, 0)),
        out_shape=jax.ShapeDtypeStruct((n * n // 128, 128), jnp.float32),
        scratch_shapes=[pltpu.VMEM((n, n), jnp.float32)],
    )


@functools.lru_cache(maxsize=None)
def _make_edge_reduce(n, e, scale):
    e_pad = ((e + _LANES - 1) // _LANES) * _LANES
    chunks = e_pad // _LANES
    mesh = plsc.VectorSubcoreMesh(core_axis_name="c", subcore_axis_name="s",
                                  num_cores=1)

    @functools.partial(
        pl.kernel,
        mesh=mesh,
        out_type=jax.ShapeDtypeStruct((_LANES,), jnp.float32),
        compiler_params=pltpu.CompilerParams(needs_layout_passes=False),
        scratch_types=[
            pltpu.VMEM((n * n,), jnp.float32),
            pltpu.VMEM((e_pad,), jnp.int32),
            pltpu.VMEM((e_pad,), jnp.int32),
            pltpu.VMEM((e_pad,), jnp.float32),
            pltpu.VMEM((_LANES,), jnp.float32),
            pltpu.SemaphoreType.DMA,
            pltpu.SemaphoreType.DMA,
            pltpu.SemaphoreType.DMA,
            pltpu.SemaphoreType.DMA,
        ],
    )
    def k(g_hbm, l_hbm, r_hbm, w_hbm, out_hbm, g_v, l_v, r_v, w_v, o_v,
          sem0, sem1, sem2, sem3):
        wid = lax.axis_index("s")

        @pl.when(wid == 0)
        def _():
            tail = pl.ds(e_pad - _LANES, _LANES)
            l_v[tail] = jnp.zeros((_LANES,), jnp.int32)
            r_v[tail] = jnp.zeros((_LANES,), jnp.int32)
            w_v[tail] = jnp.zeros((_LANES,), jnp.float32)
            cp0 = pltpu.async_copy(g_hbm, g_v, sem0)
            cp1 = pltpu.async_copy(l_hbm, l_v.at[pl.ds(0, e)], sem1)
            cp2 = pltpu.async_copy(r_hbm, r_v.at[pl.ds(0, e)], sem2)
            cp3 = pltpu.async_copy(w_hbm, w_v.at[pl.ds(0, e)], sem3)
            cp0.wait()
            cp1.wait()
            cp2.wait()
            cp3.wait()

            def step(c, acc):
                sl = pl.ds(c * _LANES, _LANES)
                li = l_v[sl]
                ri = r_v[sl]
                we = w_v[sl]
                gll = plsc.load_gather(g_v, [li * (n + 1)])
                grr = plsc.load_gather(g_v, [ri * (n + 1)])
                glr = plsc.load_gather(g_v, [li * n + ri])
                return acc + we * (gll + grr - 2.0 * glr)

            acc = lax.fori_loop(0, chunks, step, jnp.zeros((_LANES,),
                                                           jnp.float32))
            total = jnp.sum(acc) * scale
            o_v[...] = jnp.full((_LANES,), total, jnp.float32)
            pltpu.sync_copy(o_v, out_hbm)

    return k


def kernel(logits, labels, edge_weights, left_labels, right_labels):
    batch, n = logits.shape
    e = left_labels.shape[0]
    e_pad = ((e + _LANES - 1) // _LANES) * _LANES
    pad = e_pad - e
    lt = pltpu.with_memory_space_constraint(logits.T, pltpu.MemorySpace.HBM)
    bt = pltpu.with_memory_space_constraint(labels.T, pltpu.MemorySpace.HBM)
    g = _make_gram(batch, n, 8192)(lt, bt)
    scale = 0.1 / (batch * e)
    out = _make_edge_reduce(n, e, scale)(g.reshape(n * n), left_labels,
                                         right_labels, edge_weights)
    return out[0]


# speedup vs baseline: 2.6717x; 1.0016x over previous
import functools

import jax
import jax.numpy as jnp
from jax import lax
from jax.experimental import pallas as pl
from jax.experimental.pallas import tpu as pltpu
from jax.experimental.pallas import tpu_sc as plsc

_LANES = 16


def _gram_body(n, lt_ref, bt_ref, out_ref, acc_ref):
    lab = bt_ref[...]
    lgt = lt_ref[...]
    ann = (lab == 0.0) | (lab == 1.0)
    y = jnp.where(ann, 2.0 * lab - 1.0, 2.0 * jax.nn.sigmoid(lgt) - 1.0)
    g = lax.dot_general(y, y, (((1,), (1,)), ((), ())),
                        preferred_element_type=jnp.float32)

    @pl.when(pl.program_id(0) == 0)
    def _():
        acc_ref[...] = jnp.zeros_like(acc_ref)

    acc_ref[...] += g

    @pl.when(pl.program_id(0) == pl.num_programs(0) - 1)
    def _():
        gf = acc_ref[...]
        per_row = 128 // n
        for q in range(n * n // 128):
            for m in range(per_row):
                out_ref[q:q + 1, pl.ds(n * m, n)] = gf[per_row * q + m:
                                                       per_row * q + m + 1, :]


@functools.lru_cache(maxsize=None)
def _make_gram(batch, n, block):
    grid = batch // block
    return pl.pallas_call(
        functools.partial(_gram_body, n),
        grid=(grid,),
        in_specs=[
            pl.BlockSpec((n, block), lambda i: (0, i)),
            pl.BlockSpec((n, block), lambda i: (0, i)),
        ],
        out_specs=pl.BlockSpec((n * n // 128, 128), lambda i: (0, 0)),
        out_shape=jax.ShapeDtypeStruct((n * n // 128, 128), jnp.float32),
        scratch_shapes=[pltpu.VMEM((n, n), jnp.float32)],
    )


@functools.lru_cache(maxsize=None)
def _make_edge_reduce(n, e, scale):
    e_pad = ((e + _LANES - 1) // _LANES) * _LANES
    chunks = e_pad // _LANES
    mesh = plsc.VectorSubcoreMesh(core_axis_name="c", subcore_axis_name="s",
                                  num_cores=1)

    @functools.partial(
        pl.kernel,
        mesh=mesh,
        out_type=jax.ShapeDtypeStruct((_LANES,), jnp.float32),
        compiler_params=pltpu.CompilerParams(needs_layout_passes=False),
        scratch_types=[
            pltpu.VMEM((n * n,), jnp.float32),
            pltpu.VMEM((e_pad,), jnp.int32),
            pltpu.VMEM((e_pad,), jnp.int32),
            pltpu.VMEM((e_pad,), jnp.float32),
            pltpu.VMEM((_LANES,), jnp.float32),
            pltpu.SemaphoreType.DMA,
            pltpu.SemaphoreType.DMA,
            pltpu.SemaphoreType.DMA,
            pltpu.SemaphoreType.DMA,
        ],
    )
    def k(g_hbm, l_hbm, r_hbm, w_hbm, out_hbm, g_v, l_v, r_v, w_v, o_v,
          sem0, sem1, sem2, sem3):
        wid = lax.axis_index("s")

        @pl.when(wid == 0)
        def _():
            tail = pl.ds(e_pad - _LANES, _LANES)
            l_v[tail] = jnp.zeros((_LANES,), jnp.int32)
            r_v[tail] = jnp.zeros((_LANES,), jnp.int32)
            w_v[tail] = jnp.zeros((_LANES,), jnp.float32)
            cp0 = pltpu.async_copy(g_hbm, g_v, sem0)
            cp1 = pltpu.async_copy(l_hbm, l_v.at[pl.ds(0, e)], sem1)
            cp2 = pltpu.async_copy(r_hbm, r_v.at[pl.ds(0, e)], sem2)
            cp3 = pltpu.async_copy(w_hbm, w_v.at[pl.ds(0, e)], sem3)
            cp0.wait()
            cp1.wait()
            cp2.wait()
            cp3.wait()

            def step(c, acc):
                sl = pl.ds(c * _LANES, _LANES)
                li = l_v[sl]
                ri = r_v[sl]
                we = w_v[sl]
                gll = plsc.load_gather(g_v, [li * (n + 1)])
                grr = plsc.load_gather(g_v, [ri * (n + 1)])
                glr = plsc.load_gather(g_v, [li * n + ri])
                return acc + we * (gll + grr - 2.0 * glr)

            acc = lax.fori_loop(0, chunks, step, jnp.zeros((_LANES,),
                                                           jnp.float32))
            total = jnp.sum(acc) * scale
            o_v[...] = jnp.full((_LANES,), total, jnp.float32)
            pltpu.sync_copy(o_v, out_hbm)

    return k


def kernel(logits, labels, edge_weights, left_labels, right_labels):
    batch, n = logits.shape
    e = left_labels.shape[0]
    lt = pltpu.with_memory_space_constraint(logits.T, pltpu.MemorySpace.HBM)
    bt = pltpu.with_memory_space_constraint(labels.T, pltpu.MemorySpace.HBM)
    g = _make_gram(batch, n, 8192)(lt, bt)
    scale = 0.1 / (batch * e)
    out = _make_edge_reduce(n, e, scale)(g.reshape(n * n), left_labels,
                                         right_labels, edge_weights)
    return out[0]
